# Initial kernel scaffold; baseline (speedup 1.0000x reference)
import jax
import jax.numpy as jnp
from jax.experimental import pallas as pl


def kernel(x, train_mask, valid_mask, test_mask, W1, b1, W2, b2, GW1, Gb1, GW2, Gb2):
    raise NotImplementedError("write your pallas kernel here")


# speedup vs baseline: 11.1308x; 11.1308x over previous
import jax
import jax.numpy as jnp
from jax.experimental import pallas as pl

N, D, H, K = 4096, 256, 256, 16
RB = 256
NB = N // RB
BIG = 3.0e38


def _mlp_body(x_ref, w1_ref, b1_ref, w2_ref, b2_ref, gw1_ref,
              h_ref, m1_ref, sq_ref):
    x = x_ref[...]
    h1 = jnp.maximum(jnp.dot(x, w1_ref[...]) + b1_ref[...], 0.0)
    h = jnp.dot(h1, w2_ref[...]) + b2_ref[...]
    h_ref[...] = h
    m1_ref[...] = jnp.dot(x, gw1_ref[...])
    sq_ref[...] = jnp.sum(h * h, axis=1, keepdims=True)


def _knn_body(hblk_ref, sqblk_ref, h_ref, sqrow_ref,
              idx_ref, val_ref, deg_ref):
    g = jax.lax.dot_general(hblk_ref[...], h_ref[...],
                            (((1,), (1,)), ((), ())))
    d2 = sqblk_ref[...] + sqrow_ref[...] - 2.0 * g
    work = jnp.maximum(d2, 0.0)
    col = jax.lax.broadcasted_iota(jnp.int32, (RB, N), 1)
    deg = jnp.zeros((RB, 1), jnp.float32)
    for t in range(K):
        v = jnp.min(work, axis=1, keepdims=True)
        j = jnp.min(jnp.where(work == v, col, N), axis=1, keepdims=True)
        val_ref[:, t:t + 1] = v
        idx_ref[:, t:t + 1] = j
        deg = deg + v
        work = jnp.where(col == j, BIG, work)
    deg_ref[...] = deg


def _gcn1_body(idx_ref, val_ref, degb_ref, degr_ref, m1_ref, gb1_ref, gw2_ref,
               a_ref, m2_ref):
    col = jax.lax.broadcasted_iota(jnp.int32, (RB, N), 1)
    acc = jnp.zeros((RB, N), jnp.float32)
    for t in range(K):
        acc = acc + jnp.where(col == idx_ref[:, t:t + 1],
                              val_ref[:, t:t + 1], 0.0)
    a = acc / jnp.sqrt(degr_ref[...] * degb_ref[...] + 1e-8)
    a_ref[...] = a
    h1 = jnp.maximum(jnp.dot(a, m1_ref[...]) + gb1_ref[...], 0.0)
    m2_ref[...] = jnp.dot(h1, gw2_ref[...])


def _gcn2_body(a_ref, m2r_ref, gb2_ref, out_ref):
    out_ref[...] = (jnp.sum(a_ref[...] * m2r_ref[...], axis=1, keepdims=True)
                    + gb2_ref[0, 0])


def kernel(x, train_mask, valid_mask, test_mask,
           W1, b1, W2, b2, GW1, Gb1, GW2, Gb2):
    b1r = b1.reshape(1, H)
    b2r = b2.reshape(1, H)
    gb1r = Gb1.reshape(1, H)
    gb2r = Gb2.reshape(1, 1)

    h, m1, sq = pl.pallas_call(
        _mlp_body,
        grid=(NB,),
        in_specs=[
            pl.BlockSpec((RB, D), lambda i: (i, 0)),
            pl.BlockSpec((D, H), lambda i: (0, 0)),
            pl.BlockSpec((1, H), lambda i: (0, 0)),
            pl.BlockSpec((H, H), lambda i: (0, 0)),
            pl.BlockSpec((1, H), lambda i: (0, 0)),
            pl.BlockSpec((D, H), lambda i: (0, 0)),
        ],
        out_specs=[
            pl.BlockSpec((RB, H), lambda i: (i, 0)),
            pl.BlockSpec((RB, H), lambda i: (i, 0)),
            pl.BlockSpec((RB, 1), lambda i: (i, 0)),
        ],
        out_shape=[
            jax.ShapeDtypeStruct((N, H), jnp.float32),
            jax.ShapeDtypeStruct((N, H), jnp.float32),
            jax.ShapeDtypeStruct((N, 1), jnp.float32),
        ],
    )(x, W1, b1r, W2, b2r, GW1)

    sq_row = sq.reshape(1, N)
    idx, vals, deg = pl.pallas_call(
        _knn_body,
        grid=(NB,),
        in_specs=[
            pl.BlockSpec((RB, H), lambda i: (i, 0)),
            pl.BlockSpec((RB, 1), lambda i: (i, 0)),
            pl.BlockSpec((N, H), lambda i: (0, 0)),
            pl.BlockSpec((1, N), lambda i: (0, 0)),
        ],
        out_specs=[
            pl.BlockSpec((RB, K), lambda i: (i, 0)),
            pl.BlockSpec((RB, K), lambda i: (i, 0)),
            pl.BlockSpec((RB, 1), lambda i: (i, 0)),
        ],
        out_shape=[
            jax.ShapeDtypeStruct((N, K), jnp.int32),
            jax.ShapeDtypeStruct((N, K), jnp.float32),
            jax.ShapeDtypeStruct((N, 1), jnp.float32),
        ],
    )(h, sq, h, sq_row)

    deg_row = deg.reshape(1, N)
    a_mat, m2 = pl.pallas_call(
        _gcn1_body,
        grid=(NB,),
        in_specs=[
            pl.BlockSpec((RB, K), lambda i: (i, 0)),
            pl.BlockSpec((RB, K), lambda i: (i, 0)),
            pl.BlockSpec((RB, 1), lambda i: (i, 0)),
            pl.BlockSpec((1, N), lambda i: (0, 0)),
            pl.BlockSpec((N, H), lambda i: (0, 0)),
            pl.BlockSpec((1, H), lambda i: (0, 0)),
            pl.BlockSpec((H, 1), lambda i: (0, 0)),
        ],
        out_specs=[
            pl.BlockSpec((RB, N), lambda i: (i, 0)),
            pl.BlockSpec((RB, 1), lambda i: (i, 0)),
        ],
        out_shape=[
            jax.ShapeDtypeStruct((N, N), jnp.float32),
            jax.ShapeDtypeStruct((N, 1), jnp.float32),
        ],
    )(idx, vals, deg, deg_row, m1, gb1r, GW2)

    m2_row = m2.reshape(1, N)
    out = pl.pallas_call(
        _gcn2_body,
        grid=(NB,),
        in_specs=[
            pl.BlockSpec((RB, N), lambda i: (i, 0)),
            pl.BlockSpec((1, N), lambda i: (0, 0)),
            pl.BlockSpec((1, 1), lambda i: (0, 0)),
        ],
        out_specs=pl.BlockSpec((RB, 1), lambda i: (i, 0)),
        out_shape=jax.ShapeDtypeStruct((N, 1), jnp.float32),
    )(a_mat, m2_row, gb2r)

    src = idx.reshape(-1)
    dst = jnp.repeat(jnp.arange(N, dtype=jnp.int32), K)
    edge_index = jnp.stack([src, dst], axis=0)
    edge_attr = vals.reshape(-1)
    return (out, edge_index, edge_attr)


# speedup vs baseline: 13.5913x; 1.2211x over previous
import jax
import jax.numpy as jnp
from jax.experimental import pallas as pl

N, D, H, K = 4096, 256, 256, 16
RB = 256
NB = N // RB
BIG = 3.0e38


def _mlp_body(x_ref, w1_ref, b1_ref, w2_ref, b2_ref, gw1_ref,
              h_ref, m1_ref, sq_ref):
    x = x_ref[...]
    h1 = jnp.maximum(jnp.dot(x, w1_ref[...]) + b1_ref[...], 0.0)
    h = jnp.dot(h1, w2_ref[...]) + b2_ref[...]
    h_ref[...] = h
    m1_ref[...] = jnp.dot(x, gw1_ref[...])
    sq_ref[...] = jnp.sum(h * h, axis=1, keepdims=True)


def _knn_body(hblk_ref, sqblk_ref, h_ref, sqrow_ref,
              idx_ref, val_ref, deg_ref, aun_ref):
    g = jax.lax.dot_general(hblk_ref[...], h_ref[...],
                            (((1,), (1,)), ((), ())))
    d2 = jnp.maximum(sqblk_ref[...] + sqrow_ref[...] - 2.0 * g, 0.0)
    work = d2
    col = jax.lax.broadcasted_iota(jnp.int32, (RB, N), 1)
    deg = jnp.zeros((RB, 1), jnp.float32)
    for t in range(K):
        v = jnp.min(work, axis=1, keepdims=True)
        j = jnp.min(jnp.where(work == v, col, N), axis=1, keepdims=True)
        val_ref[:, t:t + 1] = v
        idx_ref[:, t:t + 1] = j
        deg = deg + v
        work = jnp.where(col == j, BIG, work)
    deg_ref[...] = deg
    aun_ref[...] = jnp.where(work != d2, d2, 0.0)


def _gcn1_body(aun_ref, rsb_ref, rsr_ref, m1_ref, gb1_ref, gw2_ref,
               m2_ref):
    m1s = m1_ref[...] * jnp.transpose(rsr_ref[...])
    agg = jax.lax.dot_general(aun_ref[...], m1s, (((1,), (0,)), ((), ())))
    h1 = jnp.maximum(agg * rsb_ref[...] + gb1_ref[...], 0.0)
    m2_ref[...] = jnp.dot(h1, gw2_ref[...])


def _gcn2_body(aun_ref, m2r_ref, rsb_ref, rsr_ref, gb2_ref, out_ref):
    m2s = m2r_ref[...] * rsr_ref[...]
    s = jnp.sum(aun_ref[...] * m2s, axis=1, keepdims=True)
    out_ref[...] = s * rsb_ref[...] + gb2_ref[0, 0]


def _rs_body(deg_ref, rs_ref):
    deg = deg_ref[...]
    rs_ref[...] = jnp.where(deg > 0.0, 1.0 / jnp.sqrt(deg), 0.0)


def kernel(x, train_mask, valid_mask, test_mask,
           W1, b1, W2, b2, GW1, Gb1, GW2, Gb2):
    b1r = b1.reshape(1, H)
    b2r = b2.reshape(1, H)
    gb1r = Gb1.reshape(1, H)
    gb2r = Gb2.reshape(1, 1)

    h, m1, sq = pl.pallas_call(
        _mlp_body,
        grid=(NB,),
        in_specs=[
            pl.BlockSpec((RB, D), lambda i: (i, 0)),
            pl.BlockSpec((D, H), lambda i: (0, 0)),
            pl.BlockSpec((1, H), lambda i: (0, 0)),
            pl.BlockSpec((H, H), lambda i: (0, 0)),
            pl.BlockSpec((1, H), lambda i: (0, 0)),
            pl.BlockSpec((D, H), lambda i: (0, 0)),
        ],
        out_specs=[
            pl.BlockSpec((RB, H), lambda i: (i, 0)),
            pl.BlockSpec((RB, H), lambda i: (i, 0)),
            pl.BlockSpec((RB, 1), lambda i: (i, 0)),
        ],
        out_shape=[
            jax.ShapeDtypeStruct((N, H), jnp.float32),
            jax.ShapeDtypeStruct((N, H), jnp.float32),
            jax.ShapeDtypeStruct((N, 1), jnp.float32),
        ],
    )(x, W1, b1r, W2, b2r, GW1)

    sq_row = sq.reshape(1, N)
    idx, vals, deg, a_un = pl.pallas_call(
        _knn_body,
        grid=(NB,),
        in_specs=[
            pl.BlockSpec((RB, H), lambda i: (i, 0)),
            pl.BlockSpec((RB, 1), lambda i: (i, 0)),
            pl.BlockSpec((N, H), lambda i: (0, 0)),
            pl.BlockSpec((1, N), lambda i: (0, 0)),
        ],
        out_specs=[
            pl.BlockSpec((RB, K), lambda i: (i, 0)),
            pl.BlockSpec((RB, K), lambda i: (i, 0)),
            pl.BlockSpec((RB, 1), lambda i: (i, 0)),
            pl.BlockSpec((RB, N), lambda i: (i, 0)),
        ],
        out_shape=[
            jax.ShapeDtypeStruct((N, K), jnp.int32),
            jax.ShapeDtypeStruct((N, K), jnp.float32),
            jax.ShapeDtypeStruct((N, 1), jnp.float32),
            jax.ShapeDtypeStruct((N, N), jnp.float32),
        ],
    )(h, sq, h, sq_row)

    rs = pl.pallas_call(
        _rs_body,
        grid=(1,),
        in_specs=[pl.BlockSpec((N, 1), lambda i: (0, 0))],
        out_specs=pl.BlockSpec((N, 1), lambda i: (0, 0)),
        out_shape=jax.ShapeDtypeStruct((N, 1), jnp.float32),
    )(deg)
    rs_row = rs.reshape(1, N)

    m2 = pl.pallas_call(
        _gcn1_body,
        grid=(NB,),
        in_specs=[
            pl.BlockSpec((RB, N), lambda i: (i, 0)),
            pl.BlockSpec((RB, 1), lambda i: (i, 0)),
            pl.BlockSpec((1, N), lambda i: (0, 0)),
            pl.BlockSpec((N, H), lambda i: (0, 0)),
            pl.BlockSpec((1, H), lambda i: (0, 0)),
            pl.BlockSpec((H, 1), lambda i: (0, 0)),
        ],
        out_specs=pl.BlockSpec((RB, 1), lambda i: (i, 0)),
        out_shape=jax.ShapeDtypeStruct((N, 1), jnp.float32),
    )(a_un, rs, rs_row, m1, gb1r, GW2)

    m2_row = m2.reshape(1, N)
    out = pl.pallas_call(
        _gcn2_body,
        grid=(NB,),
        in_specs=[
            pl.BlockSpec((RB, N), lambda i: (i, 0)),
            pl.BlockSpec((1, N), lambda i: (0, 0)),
            pl.BlockSpec((RB, 1), lambda i: (i, 0)),
            pl.BlockSpec((1, N), lambda i: (0, 0)),
            pl.BlockSpec((1, 1), lambda i: (0, 0)),
        ],
        out_specs=pl.BlockSpec((RB, 1), lambda i: (i, 0)),
        out_shape=jax.ShapeDtypeStruct((N, 1), jnp.float32),
    )(a_un, m2_row, rs, rs_row, gb2r)

    src = idx.reshape(-1)
    dst = jnp.repeat(jnp.arange(N, dtype=jnp.int32), K)
    edge_index = jnp.stack([src, dst], axis=0)
    edge_attr = vals.reshape(-1)
    return (out, edge_index, edge_attr)


# speedup vs baseline: 13.7028x; 1.0082x over previous
import jax
import jax.numpy as jnp
from jax import lax
from jax.experimental import pallas as pl
from jax.experimental.pallas import tpu as pltpu

N, D, H, K = 4096, 256, 256, 16
RB = 256
NB = N // RB
BIG = 3.0e38


def _mlp_body(x_ref, w1_ref, b1_ref, w2_ref, b2_ref, gw1_ref,
              h_ref, m1_ref, sq_ref):
    x = x_ref[...]
    h1 = jnp.maximum(jnp.dot(x, w1_ref[...]) + b1_ref[...], 0.0)
    h = jnp.dot(h1, w2_ref[...]) + b2_ref[...]
    h_ref[...] = h
    m1_ref[...] = jnp.dot(x, gw1_ref[...])
    sq_ref[...] = jnp.sum(h * h, axis=1, keepdims=True)


def _knn_body(hblk_ref, sqblk_ref, h_ref, sqrow_ref,
              idx_ref, val_ref, deg_ref, a_ref):
    g = lax.dot_general(hblk_ref[...], h_ref[...], (((1,), (1,)), ((), ())))
    d2 = jnp.maximum(sqblk_ref[...] + sqrow_ref[...] - 2.0 * g, 0.0)
    work = d2
    col = lax.broadcasted_iota(jnp.int32, (RB, N), 1)
    deg = jnp.zeros((RB, 1), jnp.float32)
    for t in range(K):
        v = jnp.min(work, axis=1, keepdims=True)
        j = jnp.min(jnp.where(work == v, col, N), axis=1, keepdims=True)
        val_ref[:, t:t + 1] = v
        idx_ref[:, t:t + 1] = j
        deg = deg + v
        work = jnp.where(col == j, BIG, work)
    deg_ref[...] = deg
    a_ref[...] = jnp.where(work == BIG, d2, 0.0)


def _rsm_body(deg_ref, m1_ref, rs_ref, m1s_ref):
    deg = deg_ref[...]
    rs = jnp.where(deg > 0.0, 1.0 / jnp.sqrt(deg), 0.0)
    rs_ref[...] = rs
    m1s_ref[...] = rs * m1_ref[...]


def _gcn1_body(a_ref, rsblk_ref, m1s_ref, gb1_ref, gw2_ref, m2_ref):
    agg = jnp.dot(a_ref[...], m1s_ref[...]) * rsblk_ref[...]
    h1 = jnp.maximum(agg + gb1_ref[...], 0.0)
    m2_ref[...] = jnp.dot(h1, gw2_ref[...])


def _gcn2_body(a_ref, rsblk_ref, rs_ref, m2_ref, gb2_ref, out_ref):
    m2s = rs_ref[...] * m2_ref[...]
    out_ref[...] = (jnp.dot(a_ref[...], m2s) * rsblk_ref[...]
                    + gb2_ref[...])


def kernel(x, train_mask, valid_mask, test_mask,
           W1, b1, W2, b2, GW1, Gb1, GW2, Gb2):
    b1r = b1.reshape(1, H)
    b2r = b2.reshape(1, H)

    h, m1, sq = pl.pallas_call(
        _mlp_body,
        grid=(NB,),
        in_specs=[
            pl.BlockSpec((RB, D), lambda i: (i, 0)),
            pl.BlockSpec((D, H), lambda i: (0, 0)),
            pl.BlockSpec((1, H), lambda i: (0, 0)),
            pl.BlockSpec((H, H), lambda i: (0, 0)),
            pl.BlockSpec((1, H), lambda i: (0, 0)),
            pl.BlockSpec((D, H), lambda i: (0, 0)),
        ],
        out_specs=[
            pl.BlockSpec((RB, H), lambda i: (i, 0)),
            pl.BlockSpec((RB, H), lambda i: (i, 0)),
            pl.BlockSpec((RB, 1), lambda i: (i, 0)),
        ],
        out_shape=[
            jax.ShapeDtypeStruct((N, H), jnp.float32),
            jax.ShapeDtypeStruct((N, H), jnp.float32),
            jax.ShapeDtypeStruct((N, 1), jnp.float32),
        ],
    )(x, W1, b1r, W2, b2r, GW1)

    sq_row = sq.reshape(1, N)
    idx, vals, deg, a_un = pl.pallas_call(
        _knn_body,
        grid=(NB,),
        in_specs=[
            pl.BlockSpec((RB, H), lambda i: (i, 0)),
            pl.BlockSpec((RB, 1), lambda i: (i, 0)),
            pl.BlockSpec((N, H), lambda i: (0, 0)),
            pl.BlockSpec((1, N), lambda i: (0, 0)),
        ],
        out_specs=[
            pl.BlockSpec((RB, K), lambda i: (i, 0)),
            pl.BlockSpec((RB, K), lambda i: (i, 0)),
            pl.BlockSpec((RB, 1), lambda i: (i, 0)),
            pl.BlockSpec((RB, N), lambda i: (i, 0)),
        ],
        out_shape=[
            jax.ShapeDtypeStruct((N, K), jnp.int32),
            jax.ShapeDtypeStruct((N, K), jnp.float32),
            jax.ShapeDtypeStruct((N, 1), jnp.float32),
            jax.ShapeDtypeStruct((N, N), jnp.float32),
        ],
    )(h, sq, h, sq_row)

    rs, m1s = pl.pallas_call(
        _rsm_body,
        grid=(1,),
        in_specs=[
            pl.BlockSpec((N, 1), lambda i: (0, 0)),
            pl.BlockSpec((N, H), lambda i: (0, 0)),
        ],
        out_specs=[
            pl.BlockSpec((N, 1), lambda i: (0, 0)),
            pl.BlockSpec((N, H), lambda i: (0, 0)),
        ],
        out_shape=[
            jax.ShapeDtypeStruct((N, 1), jnp.float32),
            jax.ShapeDtypeStruct((N, H), jnp.float32),
        ],
    )(deg, m1)

    gb1r = Gb1.reshape(1, H)
    gw2r = GW2.reshape(H, 1)
    m2 = pl.pallas_call(
        _gcn1_body,
        grid=(NB,),
        in_specs=[
            pl.BlockSpec((RB, N), lambda i: (i, 0)),
            pl.BlockSpec((RB, 1), lambda i: (i, 0)),
            pl.BlockSpec((N, H), lambda i: (0, 0)),
            pl.BlockSpec((1, H), lambda i: (0, 0)),
            pl.BlockSpec((H, 1), lambda i: (0, 0)),
        ],
        out_specs=pl.BlockSpec((RB, 1), lambda i: (i, 0)),
        out_shape=jax.ShapeDtypeStruct((N, 1), jnp.float32),
    )(a_un, rs, m1s, gb1r, gw2r)

    gb2r = jnp.broadcast_to(Gb2.reshape(-1)[:1], (1, 1))
    out = pl.pallas_call(
        _gcn2_body,
        grid=(NB,),
        in_specs=[
            pl.BlockSpec((RB, N), lambda i: (i, 0)),
            pl.BlockSpec((RB, 1), lambda i: (i, 0)),
            pl.BlockSpec((N, 1), lambda i: (0, 0)),
            pl.BlockSpec((N, 1), lambda i: (0, 0)),
            pl.BlockSpec((1, 1), lambda i: (0, 0)),
        ],
        out_specs=pl.BlockSpec((RB, 1), lambda i: (i, 0)),
        out_shape=jax.ShapeDtypeStruct((N, 1), jnp.float32),
    )(a_un, rs, rs, m2, gb2r)

    idx_flat = idx.reshape(-1)
    vals_flat = vals.reshape(-1)
    src = idx_flat
    dst = jnp.repeat(jnp.arange(N, dtype=jnp.int32), K)
    edge_index = jnp.stack([src, dst], axis=0)
    edge_attr = vals_flat
    return (out, edge_index, edge_attr)


# speedup vs baseline: 13.7358x; 1.0024x over previous
import jax
import jax.numpy as jnp
from jax import lax
from jax.experimental import pallas as pl
from jax.experimental.pallas import tpu as pltpu
from jax.experimental.pallas import tpu_sc as plsc

N, D, H, K = 4096, 256, 256, 16
RB = 256
NB = N // RB
BIG = 3.0e38

NC, NS, LANES = 2, 16, 16
NW = NC * NS
NPW = N // NW
EPW = NPW * K
GRP = NPW // LANES


def _mlp_body(x_ref, w1_ref, b1_ref, w2_ref, b2_ref, gw1_ref,
              h_ref, m1_ref, sq_ref):
    x = x_ref[...]
    h1 = jnp.maximum(jnp.dot(x, w1_ref[...]) + b1_ref[...], 0.0)
    h = jnp.dot(h1, w2_ref[...]) + b2_ref[...]
    h_ref[...] = h
    m1_ref[...] = jnp.dot(x, gw1_ref[...])
    sq_ref[...] = jnp.sum(h * h, axis=1, keepdims=True)


def _knn_body(hblk_ref, sqblk_ref, h_ref, sqrow_ref,
              idx_ref, val_ref, deg_ref, a_ref):
    g = lax.dot_general(hblk_ref[...], h_ref[...], (((1,), (1,)), ((), ())))
    d2 = jnp.maximum(sqblk_ref[...] + sqrow_ref[...] - 2.0 * g, 0.0)
    work = d2
    col = lax.broadcasted_iota(jnp.int32, (RB, N), 1)
    deg = jnp.zeros((RB, 1), jnp.float32)
    for t in range(K):
        v = jnp.min(work, axis=1, keepdims=True)
        j = jnp.min(jnp.where(work == v, col, N), axis=1, keepdims=True)
        val_ref[:, t:t + 1] = v
        idx_ref[:, t:t + 1] = j
        deg = deg + v
        work = jnp.where(col == j, BIG, work)
    deg_ref[...] = deg
    a_ref[...] = jnp.where(work == BIG, d2, 0.0)


def _rsm_body(deg_ref, m1_ref, rs_ref, m1s_ref):
    deg = deg_ref[...]
    rs = jnp.where(deg > 0.0, 1.0 / jnp.sqrt(deg), 0.0)
    rs_ref[...] = rs
    m1s_ref[...] = rs * m1_ref[...]


def _gcn1_body(a_ref, rsblk_ref, m1s_ref, gb1_ref, gw2_ref, m2_ref):
    agg = jnp.dot(a_ref[...], m1s_ref[...]) * rsblk_ref[...]
    h1 = jnp.maximum(agg + gb1_ref[...], 0.0)
    m2_ref[...] = jnp.dot(h1, gw2_ref[...])


def _sc_gcn2(idx_hbm, vals_hbm, rs_hbm, m2_hbm, gb2_hbm,
             out_hbm,
             idx_v, vals_v, rs_v, m2_v, gb2_v, out_v):
    wid = lax.axis_index("s") * NC + lax.axis_index("c")
    base = wid * NPW
    ebase = base * K
    pltpu.sync_copy(idx_hbm.at[pl.ds(ebase, EPW)], idx_v)
    pltpu.sync_copy(vals_hbm.at[pl.ds(ebase, EPW)], vals_v)
    pltpu.sync_copy(rs_hbm, rs_v)
    pltpu.sync_copy(m2_hbm, m2_v)
    pltpu.sync_copy(gb2_hbm, gb2_v)
    lane = lax.broadcasted_iota(jnp.int32, (LANES,), 0)

    def grp_body(r, carry):
        def node_body(i, ovec):
            n = r * LANES + i
            gnode = base + n
            idxs = idx_v[pl.ds(n * K, K)]
            v16 = vals_v[pl.ds(n * K, K)]
            w16 = (v16 * plsc.load_gather(rs_v, [idxs])
                   * plsc.load_gather(rs_v, [jnp.full((LANES,), gnode,
                                                      jnp.int32)]))
            s = jnp.sum(w16 * plsc.load_gather(m2_v, [idxs]))
            return jnp.where(lane == i, s, ovec)

        ovec = lax.fori_loop(0, LANES, node_body,
                             jnp.zeros((LANES,), jnp.float32))
        out_v[pl.ds(r * LANES, LANES)] = ovec + gb2_v[...]
        return carry

    lax.fori_loop(0, GRP, grp_body, 0)
    pltpu.sync_copy(out_v, out_hbm.at[pl.ds(base, NPW)])


def kernel(x, train_mask, valid_mask, test_mask,
           W1, b1, W2, b2, GW1, Gb1, GW2, Gb2):
    b1r = b1.reshape(1, H)
    b2r = b2.reshape(1, H)

    h, m1, sq = pl.pallas_call(
        _mlp_body,
        grid=(NB,),
        in_specs=[
            pl.BlockSpec((RB, D), lambda i: (i, 0)),
            pl.BlockSpec((D, H), lambda i: (0, 0)),
            pl.BlockSpec((1, H), lambda i: (0, 0)),
            pl.BlockSpec((H, H), lambda i: (0, 0)),
            pl.BlockSpec((1, H), lambda i: (0, 0)),
            pl.BlockSpec((D, H), lambda i: (0, 0)),
        ],
        out_specs=[
            pl.BlockSpec((RB, H), lambda i: (i, 0)),
            pl.BlockSpec((RB, H), lambda i: (i, 0)),
            pl.BlockSpec((RB, 1), lambda i: (i, 0)),
        ],
        out_shape=[
            jax.ShapeDtypeStruct((N, H), jnp.float32),
            jax.ShapeDtypeStruct((N, H), jnp.float32),
            jax.ShapeDtypeStruct((N, 1), jnp.float32),
        ],
    )(x, W1, b1r, W2, b2r, GW1)

    sq_row = sq.reshape(1, N)
    idx, vals, deg, a_un = pl.pallas_call(
        _knn_body,
        grid=(NB,),
        in_specs=[
            pl.BlockSpec((RB, H), lambda i: (i, 0)),
            pl.BlockSpec((RB, 1), lambda i: (i, 0)),
            pl.BlockSpec((N, H), lambda i: (0, 0)),
            pl.BlockSpec((1, N), lambda i: (0, 0)),
        ],
        out_specs=[
            pl.BlockSpec((RB, K), lambda i: (i, 0)),
            pl.BlockSpec((RB, K), lambda i: (i, 0)),
            pl.BlockSpec((RB, 1), lambda i: (i, 0)),
            pl.BlockSpec((RB, N), lambda i: (i, 0)),
        ],
        out_shape=[
            jax.ShapeDtypeStruct((N, K), jnp.int32),
            jax.ShapeDtypeStruct((N, K), jnp.float32),
            jax.ShapeDtypeStruct((N, 1), jnp.float32),
            jax.ShapeDtypeStruct((N, N), jnp.float32),
        ],
    )(h, sq, h, sq_row)

    rs, m1s = pl.pallas_call(
        _rsm_body,
        grid=(1,),
        in_specs=[
            pl.BlockSpec((N, 1), lambda i: (0, 0)),
            pl.BlockSpec((N, H), lambda i: (0, 0)),
        ],
        out_specs=[
            pl.BlockSpec((N, 1), lambda i: (0, 0)),
            pl.BlockSpec((N, H), lambda i: (0, 0)),
        ],
        out_shape=[
            jax.ShapeDtypeStruct((N, 1), jnp.float32),
            jax.ShapeDtypeStruct((N, H), jnp.float32),
        ],
    )(deg, m1)

    gb1r = Gb1.reshape(1, H)
    gw2r = GW2.reshape(H, 1)
    m2 = pl.pallas_call(
        _gcn1_body,
        grid=(NB,),
        in_specs=[
            pl.BlockSpec((RB, N), lambda i: (i, 0)),
            pl.BlockSpec((RB, 1), lambda i: (i, 0)),
            pl.BlockSpec((N, H), lambda i: (0, 0)),
            pl.BlockSpec((1, H), lambda i: (0, 0)),
            pl.BlockSpec((H, 1), lambda i: (0, 0)),
        ],
        out_specs=pl.BlockSpec((RB, 1), lambda i: (i, 0)),
        out_shape=jax.ShapeDtypeStruct((N, 1), jnp.float32),
    )(a_un, rs, m1s, gb1r, gw2r)

    idx_flat = idx.reshape(-1)
    vals_flat = vals.reshape(-1)
    mesh = plsc.VectorSubcoreMesh(core_axis_name="c", subcore_axis_name="s",
                                  num_cores=NC, num_subcores=NS)
    gb2_b = jnp.broadcast_to(Gb2.reshape(-1)[:1], (LANES,))
    out_flat = pl.kernel(
        _sc_gcn2,
        out_type=jax.ShapeDtypeStruct((N,), jnp.float32),
        mesh=mesh,
        scratch_types=[
            pltpu.VMEM((EPW,), jnp.int32),
            pltpu.VMEM((EPW,), jnp.float32),
            pltpu.VMEM((N,), jnp.float32),
            pltpu.VMEM((N,), jnp.float32),
            pltpu.VMEM((LANES,), jnp.float32),
            pltpu.VMEM((NPW,), jnp.float32),
        ],
        compiler_params=pltpu.CompilerParams(needs_layout_passes=False),
    )(idx_flat, vals_flat, rs.reshape(-1), m2.reshape(-1), gb2_b)
    out = out_flat.reshape(N, 1)

    src = idx_flat
    dst = jnp.repeat(jnp.arange(N, dtype=jnp.int32), K)
    edge_index = jnp.stack([src, dst], axis=0)
    edge_attr = vals_flat
    return (out, edge_index, edge_attr)
